# baseline (speedup 1.0000x reference)
import jax
import jax.numpy as jnp
from jax.experimental import pallas as pl
from jax.experimental.pallas import tpu as pltpu

S = 100000
B = 1024
D = 64
BETA = 8.0
LR = 0.01
S_BLK = 1024
NS = (S + S_BLK - 1) // S_BLK
NFULL = NS - 1
TAIL = S - NFULL * S_BLK
INT_MAX = 2147483647


def _stats_kernel(q_ref, w_ref, b_ref, k_ref, v_ref, kt_ref, vt_ref,
                  retr_ref, energy_ref, bi_ref, invl_ref, qp_ref,
                  acc_s, m_s, bi_s):
    j = pl.program_id(0)

    @pl.when(j == 0)
    def _init():
        qp_ref[...] = BETA * (jnp.dot(q_ref[...], w_ref[...],
                                      preferred_element_type=jnp.float32)
                              + b_ref[...])
        acc_s[...] = jnp.zeros_like(acc_s)
        m_s[...] = jnp.full_like(m_s, -jnp.inf)
        bi_s[...] = jnp.zeros_like(bi_s)

    iota = jax.lax.broadcasted_iota(jnp.int32, (B, S_BLK), 1)

    sim = jax.lax.dot_general(qp_ref[...], k_ref[...],
                              (((1,), (1,)), ((), ())),
                              preferred_element_type=jnp.float32)
    e = jnp.exp(sim)
    vaug = jnp.concatenate(
        [v_ref[...], jnp.ones((S_BLK, 1), jnp.float32)], axis=1)
    acc_s[...] += jnp.dot(e, vaug, preferred_element_type=jnp.float32)
    bm = jnp.max(sim, axis=1, keepdims=True)
    bidx = j * S_BLK + jnp.min(jnp.where(sim == bm, iota, INT_MAX), axis=1,
                               keepdims=True)
    upd = bm > m_s[...]
    m_s[...] = jnp.where(upd, bm, m_s[...])
    bi_s[...] = jnp.where(upd, bidx, bi_s[...])

    @pl.when(j == NFULL - 1)
    def _tail_and_finalize():
        simt = jax.lax.dot_general(qp_ref[...], kt_ref[...],
                                   (((1,), (1,)), ((), ())),
                                   preferred_element_type=jnp.float32)
        simt = jnp.where(iota < TAIL, simt, -jnp.inf)
        et = jnp.exp(simt)
        rowt = jax.lax.broadcasted_iota(jnp.int32, (S_BLK, D), 0)
        vt = jnp.where(rowt < TAIL, vt_ref[...], 0.0)
        vaugt = jnp.concatenate(
            [vt, jnp.ones((S_BLK, 1), jnp.float32)], axis=1)
        acc = acc_s[...] + jnp.dot(et, vaugt,
                                   preferred_element_type=jnp.float32)
        bmt = jnp.max(simt, axis=1, keepdims=True)
        bidxt = NFULL * S_BLK + jnp.min(
            jnp.where(simt == bmt, iota, INT_MAX), axis=1, keepdims=True)
        updt = bmt > m_s[...]
        bi = jnp.where(updt, bidxt, bi_s[...])

        l = acc[:, D:D + 1]
        retr_ref[...] = acc[:, :D] / l
        energy_ref[...] = -jnp.log(l)
        invl_ref[...] = 1.0 / l
        bi_ref[...] = bi


def _write_kernel(qp_ref, invl_ref, bic_ref, bir_ref, wv_ref, wr_ref,
                  k_ref, v_ref, attn_ref, nv_ref, keep_s, wvaug_s):
    j = pl.program_id(0)

    @pl.when(j == 0)
    def _init():
        colb = jax.lax.broadcasted_iota(jnp.int32, (B, B), 1)
        win = jnp.max(jnp.where(bir_ref[...] == bic_ref[...], colb, -1),
                      axis=1, keepdims=True)
        rowb = jax.lax.broadcasted_iota(jnp.int32, (B, 1), 0)
        wr_on = wr_ref[0, 0] != 0
        keep_s[...] = jnp.where((win == rowb) & wr_on, 1.0, 0.0)
        wvaug_s[...] = jnp.concatenate(
            [wv_ref[...], jnp.ones((B, 1), jnp.float32)], axis=1)

    sim = jax.lax.dot_general(qp_ref[...], k_ref[...],
                              (((1,), (1,)), ((), ())),
                              preferred_element_type=jnp.float32)
    attn_ref[...] = jnp.exp(sim) * invl_ref[...]

    iota = jax.lax.broadcasted_iota(jnp.int32, (B, S_BLK), 1)
    eqf = jnp.where(bic_ref[...] - j * S_BLK == iota, keep_s[...], 0.0)
    merged = jax.lax.dot_general(
        eqf, wvaug_s[...], (((0,), (0,)), ((), ())),
        preferred_element_type=jnp.float32)
    hasc = merged[:, D:D + 1]
    nv_ref[...] = v_ref[...] * (1.0 - LR * hasc) + LR * merged[:, :D]


def kernel(query, write_value, keys, values, W, b, write=1):
    b2 = jnp.asarray(b, jnp.float32).reshape(1, D)
    wr = jnp.asarray(write, jnp.int32).reshape(1, 1)

    retr, energy, bi, invl, qp = pl.pallas_call(
        _stats_kernel,
        grid=(NFULL,),
        in_specs=[
            pl.BlockSpec((B, D), lambda j: (0, 0)),
            pl.BlockSpec((D, D), lambda j: (0, 0)),
            pl.BlockSpec((1, D), lambda j: (0, 0)),
            pl.BlockSpec((S_BLK, D), lambda j: (j, 0)),
            pl.BlockSpec((S_BLK, D), lambda j: (j, 0)),
            pl.BlockSpec((S_BLK, D), lambda j: (NFULL, 0)),
            pl.BlockSpec((S_BLK, D), lambda j: (NFULL, 0)),
        ],
        out_specs=[
            pl.BlockSpec((B, D), lambda j: (0, 0)),
            pl.BlockSpec((B, 1), lambda j: (0, 0)),
            pl.BlockSpec((B, 1), lambda j: (0, 0)),
            pl.BlockSpec((B, 1), lambda j: (0, 0)),
            pl.BlockSpec((B, D), lambda j: (0, 0)),
        ],
        out_shape=[
            jax.ShapeDtypeStruct((B, D), jnp.float32),
            jax.ShapeDtypeStruct((B, 1), jnp.float32),
            jax.ShapeDtypeStruct((B, 1), jnp.int32),
            jax.ShapeDtypeStruct((B, 1), jnp.float32),
            jax.ShapeDtypeStruct((B, D), jnp.float32),
        ],
        scratch_shapes=[
            pltpu.VMEM((B, D + 1), jnp.float32),
            pltpu.VMEM((B, 1), jnp.float32),
            pltpu.VMEM((B, 1), jnp.int32),
        ],
        compiler_params=pltpu.CompilerParams(
            dimension_semantics=("arbitrary",),
        ),
    )(query, W, b2, keys, values, keys, values)

    attn, nv = pl.pallas_call(
        _write_kernel,
        grid=(NS,),
        in_specs=[
            pl.BlockSpec((B, D), lambda j: (0, 0)),
            pl.BlockSpec((B, 1), lambda j: (0, 0)),
            pl.BlockSpec((B, 1), lambda j: (0, 0)),
            pl.BlockSpec((1, B), lambda j: (0, 0)),
            pl.BlockSpec((B, D), lambda j: (0, 0)),
            pl.BlockSpec((1, 1), lambda j: (0, 0)),
            pl.BlockSpec((S_BLK, D), lambda j: (j, 0)),
            pl.BlockSpec((S_BLK, D), lambda j: (j, 0)),
        ],
        out_specs=[
            pl.BlockSpec((B, S_BLK), lambda j: (0, j)),
            pl.BlockSpec((S_BLK, D), lambda j: (j, 0)),
        ],
        out_shape=[
            jax.ShapeDtypeStruct((B, S), jnp.float32),
            jax.ShapeDtypeStruct((S, D), jnp.float32),
        ],
        scratch_shapes=[
            pltpu.VMEM((B, 1), jnp.float32),
            pltpu.VMEM((B, D + 1), jnp.float32),
        ],
        compiler_params=pltpu.CompilerParams(
            dimension_semantics=("arbitrary",),
        ),
    )(qp, invl, bi, bi.reshape(1, B), write_value, wr, keys, values)

    return retr, attn, energy.reshape(B), nv


# speedup vs baseline: 2.3579x; 2.3579x over previous
import jax
import jax.numpy as jnp
from jax.experimental import pallas as pl
from jax.experimental.pallas import tpu as pltpu

S = 100000
B = 1024
D = 64
BETA = 8.0
LR = 0.01
S_BLK = 1024
NS = (S + S_BLK - 1) // S_BLK
NFULL = NS - 1
TAIL = S - NFULL * S_BLK
INT_MAX = 2147483647


def _stats_kernel(qt_ref, wt_ref, bt_ref, k_ref, v_ref, kt_ref, vt_ref,
                  retr_ref, energy_ref, bi_ref, invl_ref, qp_ref,
                  acc_s, m_s, bi_s):
    j = pl.program_id(0)

    @pl.when(j == 0)
    def _init():
        qp_ref[...] = BETA * (
            jax.lax.dot_general(wt_ref[...], qt_ref[...],
                                (((1,), (0,)), ((), ())),
                                preferred_element_type=jnp.float32)
            + bt_ref[...])
        acc_s[...] = jnp.zeros_like(acc_s)
        m_s[...] = jnp.full_like(m_s, -jnp.inf)
        bi_s[...] = jnp.zeros_like(bi_s)

    iota = jax.lax.broadcasted_iota(jnp.int32, (S_BLK, B), 0)

    simt = jax.lax.dot_general(k_ref[...], qp_ref[...],
                               (((0,), (0,)), ((), ())),
                               preferred_element_type=jnp.float32)
    e = jnp.exp(simt)
    vaug = jnp.concatenate(
        [v_ref[...], jnp.ones((1, S_BLK), jnp.float32)], axis=0)
    acc_s[...] += jax.lax.dot_general(vaug, e, (((1,), (0,)), ((), ())),
                                      preferred_element_type=jnp.float32)
    bm = jnp.max(simt, axis=0, keepdims=True)
    bidx = j * S_BLK + jnp.min(jnp.where(simt == bm, iota, INT_MAX), axis=0,
                               keepdims=True)
    upd = bm > m_s[...]
    m_s[...] = jnp.where(upd, bm, m_s[...])
    bi_s[...] = jnp.where(upd, bidx, bi_s[...])

    @pl.when(j == NFULL - 1)
    def _tail_and_finalize():
        simt2 = jax.lax.dot_general(kt_ref[...], qp_ref[...],
                                    (((0,), (0,)), ((), ())),
                                    preferred_element_type=jnp.float32)
        simt2 = jnp.where(iota < TAIL, simt2, -jnp.inf)
        et = jnp.exp(simt2)
        colt = jax.lax.broadcasted_iota(jnp.int32, (D, S_BLK), 1)
        vt = jnp.where(colt < TAIL, vt_ref[...], 0.0)
        vaugt = jnp.concatenate(
            [vt, jnp.ones((1, S_BLK), jnp.float32)], axis=0)
        acc = acc_s[...] + jax.lax.dot_general(
            vaugt, et, (((1,), (0,)), ((), ())),
            preferred_element_type=jnp.float32)
        bmt = jnp.max(simt2, axis=0, keepdims=True)
        bidxt = NFULL * S_BLK + jnp.min(
            jnp.where(simt2 == bmt, iota, INT_MAX), axis=0, keepdims=True)
        updt = bmt > m_s[...]
        bi = jnp.where(updt, bidxt, bi_s[...])

        l = acc[D:D + 1, :]
        retr_ref[...] = acc[:D, :] / l
        energy_ref[...] = -jnp.log(l)
        invl_ref[...] = 1.0 / l
        bi_ref[...] = bi


def _write_kernel(qp_ref, invl_ref, bic_ref, bir_ref, wvt_ref, wr_ref,
                  k_ref, v_ref, attn_ref, nv_ref, keep_s, wvaug_s):
    j = pl.program_id(0)

    @pl.when(j == 0)
    def _init():
        eq2 = bic_ref[...] == bir_ref[...]
        subb = jax.lax.broadcasted_iota(jnp.int32, (B, B), 0)
        win = jnp.max(jnp.where(eq2, subb, -1), axis=0, keepdims=True)
        laneb = jax.lax.broadcasted_iota(jnp.int32, (1, B), 1)
        wr_on = wr_ref[0, 0] != 0
        keep_s[...] = jnp.where((win == laneb) & wr_on, 1.0, 0.0)
        wvaug_s[...] = jnp.concatenate(
            [wvt_ref[...], jnp.ones((1, B), jnp.float32)], axis=0)

    simt = jax.lax.dot_general(k_ref[...], qp_ref[...],
                               (((0,), (0,)), ((), ())),
                               preferred_element_type=jnp.float32)
    attn_ref[...] = jnp.exp(simt) * invl_ref[...]

    iota = jax.lax.broadcasted_iota(jnp.int32, (S_BLK, B), 0)
    eqf = jnp.where(bir_ref[...] - j * S_BLK == iota, keep_s[...], 0.0)
    merged = jax.lax.dot_general(
        wvaug_s[...], eqf, (((1,), (1,)), ((), ())),
        preferred_element_type=jnp.float32)
    hasc = merged[D:D + 1, :]
    nv_ref[...] = v_ref[...] * (1.0 - LR * hasc) + LR * merged[:D, :]


def kernel(query, write_value, keys, values, W, b, write=1):
    qt = query.T
    wvt = write_value.T
    kt = keys.T
    vt = values.T
    wt = W.T
    bt = jnp.asarray(b, jnp.float32).reshape(D, 1)
    wr = jnp.asarray(write, jnp.int32).reshape(1, 1)

    retr_t, energy_t, bi_t, invl_t, qp_t = pl.pallas_call(
        _stats_kernel,
        grid=(NFULL,),
        in_specs=[
            pl.BlockSpec((D, B), lambda j: (0, 0)),
            pl.BlockSpec((D, D), lambda j: (0, 0)),
            pl.BlockSpec((D, 1), lambda j: (0, 0)),
            pl.BlockSpec((D, S_BLK), lambda j: (0, j)),
            pl.BlockSpec((D, S_BLK), lambda j: (0, j)),
            pl.BlockSpec((D, S_BLK), lambda j: (0, NFULL)),
            pl.BlockSpec((D, S_BLK), lambda j: (0, NFULL)),
        ],
        out_specs=[
            pl.BlockSpec((D, B), lambda j: (0, 0)),
            pl.BlockSpec((1, B), lambda j: (0, 0)),
            pl.BlockSpec((1, B), lambda j: (0, 0)),
            pl.BlockSpec((1, B), lambda j: (0, 0)),
            pl.BlockSpec((D, B), lambda j: (0, 0)),
        ],
        out_shape=[
            jax.ShapeDtypeStruct((D, B), jnp.float32),
            jax.ShapeDtypeStruct((1, B), jnp.float32),
            jax.ShapeDtypeStruct((1, B), jnp.int32),
            jax.ShapeDtypeStruct((1, B), jnp.float32),
            jax.ShapeDtypeStruct((D, B), jnp.float32),
        ],
        scratch_shapes=[
            pltpu.VMEM((D + 1, B), jnp.float32),
            pltpu.VMEM((1, B), jnp.float32),
            pltpu.VMEM((1, B), jnp.int32),
        ],
        compiler_params=pltpu.CompilerParams(
            dimension_semantics=("arbitrary",),
        ),
    )(qt, wt, bt, kt, vt, kt, vt)

    attn_t, nv_t = pl.pallas_call(
        _write_kernel,
        grid=(NS,),
        in_specs=[
            pl.BlockSpec((D, B), lambda j: (0, 0)),
            pl.BlockSpec((1, B), lambda j: (0, 0)),
            pl.BlockSpec((B, 1), lambda j: (0, 0)),
            pl.BlockSpec((1, B), lambda j: (0, 0)),
            pl.BlockSpec((D, B), lambda j: (0, 0)),
            pl.BlockSpec((1, 1), lambda j: (0, 0)),
            pl.BlockSpec((D, S_BLK), lambda j: (0, j)),
            pl.BlockSpec((D, S_BLK), lambda j: (0, j)),
        ],
        out_specs=[
            pl.BlockSpec((S_BLK, B), lambda j: (j, 0)),
            pl.BlockSpec((D, S_BLK), lambda j: (0, j)),
        ],
        out_shape=[
            jax.ShapeDtypeStruct((S, B), jnp.float32),
            jax.ShapeDtypeStruct((D, S), jnp.float32),
        ],
        scratch_shapes=[
            pltpu.VMEM((1, B), jnp.float32),
            pltpu.VMEM((D + 1, B), jnp.float32),
        ],
        compiler_params=pltpu.CompilerParams(
            dimension_semantics=("arbitrary",),
        ),
    )(qp_t, invl_t, bi_t.reshape(B, 1), bi_t, wvt, wr, kt, vt)

    return retr_t.T, attn_t.T, energy_t.reshape(B), nv_t.T


# speedup vs baseline: 2.5666x; 1.0885x over previous
import jax
import jax.numpy as jnp
from jax.experimental import pallas as pl
from jax.experimental.pallas import tpu as pltpu

S = 100000
B = 1024
D = 64
BETA = 8.0
LR = 0.01
S_BLK = 2048
NS = (S + S_BLK - 1) // S_BLK
NFULL = NS - 1
TAIL = S - NFULL * S_BLK
INT_MAX = 2147483647


def _stats_kernel(qt_ref, wt_ref, bt_ref, k_ref, v_ref, kt_ref, vt_ref,
                  retr_ref, energy_ref, bi_ref, invl_ref, qp_ref,
                  acc_s, m_s, bi_s):
    j = pl.program_id(0)

    @pl.when(j == 0)
    def _init():
        qp_ref[...] = BETA * (
            jax.lax.dot_general(wt_ref[...], qt_ref[...],
                                (((1,), (0,)), ((), ())),
                                preferred_element_type=jnp.float32)
            + bt_ref[...])
        acc_s[...] = jnp.zeros_like(acc_s)
        m_s[...] = jnp.full_like(m_s, -jnp.inf)
        bi_s[...] = jnp.zeros_like(bi_s)

    iota = jax.lax.broadcasted_iota(jnp.int32, (S_BLK, B), 0)

    simt = jax.lax.dot_general(k_ref[...], qp_ref[...],
                               (((0,), (0,)), ((), ())),
                               preferred_element_type=jnp.float32)
    e = jnp.exp(simt)
    vaug = jnp.concatenate(
        [v_ref[...], jnp.ones((1, S_BLK), jnp.float32)], axis=0)
    acc_s[...] += jax.lax.dot_general(vaug, e, (((1,), (0,)), ((), ())),
                                      preferred_element_type=jnp.float32)
    bm = jnp.max(simt, axis=0, keepdims=True)
    bidx = j * S_BLK + jnp.min(jnp.where(simt == bm, iota, INT_MAX), axis=0,
                               keepdims=True)
    upd = bm > m_s[...]
    m_s[...] = jnp.where(upd, bm, m_s[...])
    bi_s[...] = jnp.where(upd, bidx, bi_s[...])

    @pl.when(j == NFULL - 1)
    def _tail_and_finalize():
        simt2 = jax.lax.dot_general(kt_ref[...], qp_ref[...],
                                    (((0,), (0,)), ((), ())),
                                    preferred_element_type=jnp.float32)
        simt2 = jnp.where(iota < TAIL, simt2, -jnp.inf)
        et = jnp.exp(simt2)
        colt = jax.lax.broadcasted_iota(jnp.int32, (D, S_BLK), 1)
        vt = jnp.where(colt < TAIL, vt_ref[...], 0.0)
        vaugt = jnp.concatenate(
            [vt, jnp.ones((1, S_BLK), jnp.float32)], axis=0)
        acc = acc_s[...] + jax.lax.dot_general(
            vaugt, et, (((1,), (0,)), ((), ())),
            preferred_element_type=jnp.float32)
        bmt = jnp.max(simt2, axis=0, keepdims=True)
        bidxt = NFULL * S_BLK + jnp.min(
            jnp.where(simt2 == bmt, iota, INT_MAX), axis=0, keepdims=True)
        updt = bmt > m_s[...]
        bi = jnp.where(updt, bidxt, bi_s[...])

        l = acc[D:D + 1, :]
        retr_ref[...] = acc[:D, :] / l
        energy_ref[...] = -jnp.log(l)
        invl_ref[...] = 1.0 / l
        bi_ref[...] = bi


def _write_kernel(qp_ref, invl_ref, bic_ref, bir_ref, wvt_ref, wr_ref,
                  k_ref, v_ref, attn_ref, nv_ref, keep_s, wvaug_s):
    j = pl.program_id(0)

    @pl.when(j == 0)
    def _init():
        eq2 = bic_ref[...] == bir_ref[...]
        subb = jax.lax.broadcasted_iota(jnp.int32, (B, B), 0)
        win = jnp.max(jnp.where(eq2, subb, -1), axis=0, keepdims=True)
        laneb = jax.lax.broadcasted_iota(jnp.int32, (1, B), 1)
        wr_on = wr_ref[0, 0] != 0
        keep_s[...] = jnp.where((win == laneb) & wr_on, 1.0, 0.0)
        wvaug_s[...] = jnp.concatenate(
            [wvt_ref[...], jnp.ones((1, B), jnp.float32)], axis=0)

    simt = jax.lax.dot_general(k_ref[...], qp_ref[...],
                               (((0,), (0,)), ((), ())),
                               preferred_element_type=jnp.float32)
    attn_ref[...] = jnp.exp(simt) * invl_ref[...]

    iota = jax.lax.broadcasted_iota(jnp.int32, (S_BLK, B), 0)
    eqf = jnp.where(bir_ref[...] - j * S_BLK == iota, keep_s[...], 0.0)
    merged = jax.lax.dot_general(
        wvaug_s[...], eqf, (((1,), (1,)), ((), ())),
        preferred_element_type=jnp.float32)
    hasc = merged[D:D + 1, :]
    nv_ref[...] = v_ref[...] * (1.0 - LR * hasc) + LR * merged[:D, :]


def kernel(query, write_value, keys, values, W, b, write=1):
    qt = query.T
    wvt = write_value.T
    kt = keys.T
    vt = values.T
    wt = W.T
    bt = jnp.asarray(b, jnp.float32).reshape(D, 1)
    wr = jnp.asarray(write, jnp.int32).reshape(1, 1)

    retr_t, energy_t, bi_t, invl_t, qp_t = pl.pallas_call(
        _stats_kernel,
        grid=(NFULL,),
        in_specs=[
            pl.BlockSpec((D, B), lambda j: (0, 0)),
            pl.BlockSpec((D, D), lambda j: (0, 0)),
            pl.BlockSpec((D, 1), lambda j: (0, 0)),
            pl.BlockSpec((D, S_BLK), lambda j: (0, j)),
            pl.BlockSpec((D, S_BLK), lambda j: (0, j)),
            pl.BlockSpec((D, S_BLK), lambda j: (0, NFULL)),
            pl.BlockSpec((D, S_BLK), lambda j: (0, NFULL)),
        ],
        out_specs=[
            pl.BlockSpec((D, B), lambda j: (0, 0)),
            pl.BlockSpec((1, B), lambda j: (0, 0)),
            pl.BlockSpec((1, B), lambda j: (0, 0)),
            pl.BlockSpec((1, B), lambda j: (0, 0)),
            pl.BlockSpec((D, B), lambda j: (0, 0)),
        ],
        out_shape=[
            jax.ShapeDtypeStruct((D, B), jnp.float32),
            jax.ShapeDtypeStruct((1, B), jnp.float32),
            jax.ShapeDtypeStruct((1, B), jnp.int32),
            jax.ShapeDtypeStruct((1, B), jnp.float32),
            jax.ShapeDtypeStruct((D, B), jnp.float32),
        ],
        scratch_shapes=[
            pltpu.VMEM((D + 1, B), jnp.float32),
            pltpu.VMEM((1, B), jnp.float32),
            pltpu.VMEM((1, B), jnp.int32),
        ],
        compiler_params=pltpu.CompilerParams(
            dimension_semantics=("arbitrary",),
        ),
    )(qt, wt, bt, kt, vt, kt, vt)

    attn_t, nv_t = pl.pallas_call(
        _write_kernel,
        grid=(NS,),
        in_specs=[
            pl.BlockSpec((D, B), lambda j: (0, 0)),
            pl.BlockSpec((1, B), lambda j: (0, 0)),
            pl.BlockSpec((B, 1), lambda j: (0, 0)),
            pl.BlockSpec((1, B), lambda j: (0, 0)),
            pl.BlockSpec((D, B), lambda j: (0, 0)),
            pl.BlockSpec((1, 1), lambda j: (0, 0)),
            pl.BlockSpec((D, S_BLK), lambda j: (0, j)),
            pl.BlockSpec((D, S_BLK), lambda j: (0, j)),
        ],
        out_specs=[
            pl.BlockSpec((S_BLK, B), lambda j: (j, 0)),
            pl.BlockSpec((D, S_BLK), lambda j: (0, j)),
        ],
        out_shape=[
            jax.ShapeDtypeStruct((S, B), jnp.float32),
            jax.ShapeDtypeStruct((D, S), jnp.float32),
        ],
        scratch_shapes=[
            pltpu.VMEM((1, B), jnp.float32),
            pltpu.VMEM((D + 1, B), jnp.float32),
        ],
        compiler_params=pltpu.CompilerParams(
            dimension_semantics=("arbitrary",),
        ),
    )(qp_t, invl_t, bi_t.reshape(B, 1), bi_t, wvt, wr, kt, vt)

    return retr_t.T, attn_t.T, energy_t.reshape(B), nv_t.T


# speedup vs baseline: 2.5884x; 1.0085x over previous
import jax
import jax.numpy as jnp
from jax.experimental import pallas as pl
from jax.experimental.pallas import tpu as pltpu

S = 100000
B = 1024
D = 64
BETA = 8.0
LR = 0.01
S_BLK = 2560
NS = (S + S_BLK - 1) // S_BLK
NFULL = NS - 1
TAIL = S - NFULL * S_BLK
INT_MAX = 2147483647


def _stats_kernel(qt_ref, wt_ref, bt_ref, k_ref, v_ref, kt_ref, vt_ref,
                  retr_ref, energy_ref, bi_ref, invl_ref, qp_ref,
                  acc_s, m_s, bi_s):
    j = pl.program_id(0)

    @pl.when(j == 0)
    def _init():
        qp_ref[...] = BETA * (
            jax.lax.dot_general(wt_ref[...], qt_ref[...],
                                (((1,), (0,)), ((), ())),
                                preferred_element_type=jnp.float32)
            + bt_ref[...])
        acc_s[...] = jnp.zeros_like(acc_s)
        m_s[...] = jnp.full_like(m_s, -jnp.inf)
        bi_s[...] = jnp.zeros_like(bi_s)

    iota = jax.lax.broadcasted_iota(jnp.int32, (S_BLK, B), 0)

    simt = jax.lax.dot_general(k_ref[...], qp_ref[...],
                               (((0,), (0,)), ((), ())),
                               preferred_element_type=jnp.float32)
    e = jnp.exp(simt)
    vaug = jnp.concatenate(
        [v_ref[...], jnp.ones((1, S_BLK), jnp.float32)], axis=0)
    acc_s[...] += jax.lax.dot_general(vaug, e, (((1,), (0,)), ((), ())),
                                      preferred_element_type=jnp.float32)
    bm = jnp.max(simt, axis=0, keepdims=True)
    bidx = j * S_BLK + jnp.min(jnp.where(simt == bm, iota, INT_MAX), axis=0,
                               keepdims=True)
    upd = bm > m_s[...]
    m_s[...] = jnp.where(upd, bm, m_s[...])
    bi_s[...] = jnp.where(upd, bidx, bi_s[...])

    @pl.when(j == NFULL - 1)
    def _tail_and_finalize():
        simt2 = jax.lax.dot_general(kt_ref[...], qp_ref[...],
                                    (((0,), (0,)), ((), ())),
                                    preferred_element_type=jnp.float32)
        simt2 = jnp.where(iota < TAIL, simt2, -jnp.inf)
        et = jnp.exp(simt2)
        colt = jax.lax.broadcasted_iota(jnp.int32, (D, S_BLK), 1)
        vt = jnp.where(colt < TAIL, vt_ref[...], 0.0)
        vaugt = jnp.concatenate(
            [vt, jnp.ones((1, S_BLK), jnp.float32)], axis=0)
        acc = acc_s[...] + jax.lax.dot_general(
            vaugt, et, (((1,), (0,)), ((), ())),
            preferred_element_type=jnp.float32)
        bmt = jnp.max(simt2, axis=0, keepdims=True)
        bidxt = NFULL * S_BLK + jnp.min(
            jnp.where(simt2 == bmt, iota, INT_MAX), axis=0, keepdims=True)
        updt = bmt > m_s[...]
        bi = jnp.where(updt, bidxt, bi_s[...])

        l = acc[D:D + 1, :]
        retr_ref[...] = acc[:D, :] / l
        energy_ref[...] = -jnp.log(l)
        invl_ref[...] = 1.0 / l
        bi_ref[...] = bi


def _write_kernel(qp_ref, invl_ref, bic_ref, bir_ref, wvt_ref, wr_ref,
                  k_ref, v_ref, attn_ref, nv_ref, keep_s, wvaug_s):
    j = pl.program_id(0)

    @pl.when(j == 0)
    def _init():
        eq2 = bic_ref[...] == bir_ref[...]
        subb = jax.lax.broadcasted_iota(jnp.int32, (B, B), 0)
        win = jnp.max(jnp.where(eq2, subb, -1), axis=0, keepdims=True)
        laneb = jax.lax.broadcasted_iota(jnp.int32, (1, B), 1)
        wr_on = wr_ref[0, 0] != 0
        keep_s[...] = jnp.where((win == laneb) & wr_on, 1.0, 0.0)
        wvaug_s[...] = jnp.concatenate(
            [wvt_ref[...], jnp.ones((1, B), jnp.float32)], axis=0)

    simt = jax.lax.dot_general(k_ref[...], qp_ref[...],
                               (((0,), (0,)), ((), ())),
                               preferred_element_type=jnp.float32)
    attn_ref[...] = jnp.exp(simt) * invl_ref[...]

    iota = jax.lax.broadcasted_iota(jnp.int32, (S_BLK, B), 0)
    eqf = jnp.where(bir_ref[...] - j * S_BLK == iota, keep_s[...], 0.0)
    merged = jax.lax.dot_general(
        wvaug_s[...], eqf, (((1,), (1,)), ((), ())),
        preferred_element_type=jnp.float32)
    hasc = merged[D:D + 1, :]
    nv_ref[...] = v_ref[...] * (1.0 - LR * hasc) + LR * merged[:D, :]


def kernel(query, write_value, keys, values, W, b, write=1):
    qt = query.T
    wvt = write_value.T
    kt = keys.T
    vt = values.T
    wt = W.T
    bt = jnp.asarray(b, jnp.float32).reshape(D, 1)
    wr = jnp.asarray(write, jnp.int32).reshape(1, 1)

    retr_t, energy_t, bi_t, invl_t, qp_t = pl.pallas_call(
        _stats_kernel,
        grid=(NFULL,),
        in_specs=[
            pl.BlockSpec((D, B), lambda j: (0, 0)),
            pl.BlockSpec((D, D), lambda j: (0, 0)),
            pl.BlockSpec((D, 1), lambda j: (0, 0)),
            pl.BlockSpec((D, S_BLK), lambda j: (0, j)),
            pl.BlockSpec((D, S_BLK), lambda j: (0, j)),
            pl.BlockSpec((D, S_BLK), lambda j: (0, NFULL)),
            pl.BlockSpec((D, S_BLK), lambda j: (0, NFULL)),
        ],
        out_specs=[
            pl.BlockSpec((D, B), lambda j: (0, 0)),
            pl.BlockSpec((1, B), lambda j: (0, 0)),
            pl.BlockSpec((1, B), lambda j: (0, 0)),
            pl.BlockSpec((1, B), lambda j: (0, 0)),
            pl.BlockSpec((D, B), lambda j: (0, 0)),
        ],
        out_shape=[
            jax.ShapeDtypeStruct((D, B), jnp.float32),
            jax.ShapeDtypeStruct((1, B), jnp.float32),
            jax.ShapeDtypeStruct((1, B), jnp.int32),
            jax.ShapeDtypeStruct((1, B), jnp.float32),
            jax.ShapeDtypeStruct((D, B), jnp.float32),
        ],
        scratch_shapes=[
            pltpu.VMEM((D + 1, B), jnp.float32),
            pltpu.VMEM((1, B), jnp.float32),
            pltpu.VMEM((1, B), jnp.int32),
        ],
        compiler_params=pltpu.CompilerParams(
            dimension_semantics=("arbitrary",),
        ),
    )(qt, wt, bt, kt, vt, kt, vt)

    attn_t, nv_t = pl.pallas_call(
        _write_kernel,
        grid=(NS,),
        in_specs=[
            pl.BlockSpec((D, B), lambda j: (0, 0)),
            pl.BlockSpec((1, B), lambda j: (0, 0)),
            pl.BlockSpec((B, 1), lambda j: (0, 0)),
            pl.BlockSpec((1, B), lambda j: (0, 0)),
            pl.BlockSpec((D, B), lambda j: (0, 0)),
            pl.BlockSpec((1, 1), lambda j: (0, 0)),
            pl.BlockSpec((D, S_BLK), lambda j: (0, j)),
            pl.BlockSpec((D, S_BLK), lambda j: (0, j)),
        ],
        out_specs=[
            pl.BlockSpec((S_BLK, B), lambda j: (j, 0)),
            pl.BlockSpec((D, S_BLK), lambda j: (0, j)),
        ],
        out_shape=[
            jax.ShapeDtypeStruct((S, B), jnp.float32),
            jax.ShapeDtypeStruct((D, S), jnp.float32),
        ],
        scratch_shapes=[
            pltpu.VMEM((1, B), jnp.float32),
            pltpu.VMEM((D + 1, B), jnp.float32),
        ],
        compiler_params=pltpu.CompilerParams(
            dimension_semantics=("arbitrary",),
        ),
    )(qp_t, invl_t, bi_t.reshape(B, 1), bi_t, wvt, wr, kt, vt)

    return retr_t.T, attn_t.T, energy_t.reshape(B), nv_t.T


# speedup vs baseline: 2.6674x; 1.0305x over previous
import jax
import jax.numpy as jnp
from jax.experimental import pallas as pl
from jax.experimental.pallas import tpu as pltpu

S = 100000
B = 1024
D = 64
BETA = 8.0
LR = 0.01
S_BLK = 4096
NS = (S + S_BLK - 1) // S_BLK
NFULL = NS - 1
TAIL = S - NFULL * S_BLK
INT_MAX = 2147483647


def _stats_kernel(qt_ref, wt_ref, bt_ref, k_ref, v_ref, kt_ref, vt_ref,
                  retr_ref, energy_ref, bi_ref, invl_ref, qp_ref,
                  acc_s, m_s, bi_s):
    j = pl.program_id(0)

    @pl.when(j == 0)
    def _init():
        qp_ref[...] = BETA * (
            jax.lax.dot_general(wt_ref[...], qt_ref[...],
                                (((1,), (0,)), ((), ())),
                                preferred_element_type=jnp.float32)
            + bt_ref[...])
        acc_s[...] = jnp.zeros_like(acc_s)
        m_s[...] = jnp.full_like(m_s, -jnp.inf)
        bi_s[...] = jnp.zeros_like(bi_s)

    iota = jax.lax.broadcasted_iota(jnp.int32, (S_BLK, B), 0)

    simt = jax.lax.dot_general(k_ref[...], qp_ref[...],
                               (((0,), (0,)), ((), ())),
                               preferred_element_type=jnp.float32)
    e = jnp.exp(simt)
    vaug = jnp.concatenate(
        [v_ref[...], jnp.ones((1, S_BLK), jnp.float32)], axis=0)
    acc_s[...] += jax.lax.dot_general(vaug, e, (((1,), (0,)), ((), ())),
                                      preferred_element_type=jnp.float32)
    bm = jnp.max(simt, axis=0, keepdims=True)
    bidx = j * S_BLK + jnp.min(jnp.where(simt == bm, iota, INT_MAX), axis=0,
                               keepdims=True)
    upd = bm > m_s[...]
    m_s[...] = jnp.where(upd, bm, m_s[...])
    bi_s[...] = jnp.where(upd, bidx, bi_s[...])

    @pl.when(j == NFULL - 1)
    def _tail_and_finalize():
        simt2 = jax.lax.dot_general(kt_ref[...], qp_ref[...],
                                    (((0,), (0,)), ((), ())),
                                    preferred_element_type=jnp.float32)
        simt2 = jnp.where(iota < TAIL, simt2, -jnp.inf)
        et = jnp.exp(simt2)
        colt = jax.lax.broadcasted_iota(jnp.int32, (D, S_BLK), 1)
        vt = jnp.where(colt < TAIL, vt_ref[...], 0.0)
        vaugt = jnp.concatenate(
            [vt, jnp.ones((1, S_BLK), jnp.float32)], axis=0)
        acc = acc_s[...] + jax.lax.dot_general(
            vaugt, et, (((1,), (0,)), ((), ())),
            preferred_element_type=jnp.float32)
        bmt = jnp.max(simt2, axis=0, keepdims=True)
        bidxt = NFULL * S_BLK + jnp.min(
            jnp.where(simt2 == bmt, iota, INT_MAX), axis=0, keepdims=True)
        updt = bmt > m_s[...]
        bi = jnp.where(updt, bidxt, bi_s[...])

        l = acc[D:D + 1, :]
        retr_ref[...] = acc[:D, :] / l
        energy_ref[...] = -jnp.log(l)
        invl_ref[...] = 1.0 / l
        bi_ref[...] = bi


def _write_kernel(qp_ref, invl_ref, bic_ref, bir_ref, wvt_ref, wr_ref,
                  k_ref, v_ref, attn_ref, nv_ref, keep_s, wvaug_s):
    j = pl.program_id(0)

    @pl.when(j == 0)
    def _init():
        eq2 = bic_ref[...] == bir_ref[...]
        subb = jax.lax.broadcasted_iota(jnp.int32, (B, B), 0)
        win = jnp.max(jnp.where(eq2, subb, -1), axis=0, keepdims=True)
        laneb = jax.lax.broadcasted_iota(jnp.int32, (1, B), 1)
        wr_on = wr_ref[0, 0] != 0
        keep_s[...] = jnp.where((win == laneb) & wr_on, 1.0, 0.0)
        wvaug_s[...] = jnp.concatenate(
            [wvt_ref[...], jnp.ones((1, B), jnp.float32)], axis=0)

    simt = jax.lax.dot_general(k_ref[...], qp_ref[...],
                               (((0,), (0,)), ((), ())),
                               preferred_element_type=jnp.float32)
    attn_ref[...] = jnp.exp(simt) * invl_ref[...]

    iota = jax.lax.broadcasted_iota(jnp.int32, (S_BLK, B), 0)
    eqf = jnp.where(bir_ref[...] - j * S_BLK == iota, keep_s[...], 0.0)
    merged = jax.lax.dot_general(
        wvaug_s[...], eqf, (((1,), (1,)), ((), ())),
        preferred_element_type=jnp.float32)
    hasc = merged[D:D + 1, :]
    nv_ref[...] = v_ref[...] * (1.0 - LR * hasc) + LR * merged[:D, :]


def kernel(query, write_value, keys, values, W, b, write=1):
    qt = query.T
    wvt = write_value.T
    kt = keys.T
    vt = values.T
    wt = W.T
    bt = jnp.asarray(b, jnp.float32).reshape(D, 1)
    wr = jnp.asarray(write, jnp.int32).reshape(1, 1)

    retr_t, energy_t, bi_t, invl_t, qp_t = pl.pallas_call(
        _stats_kernel,
        grid=(NFULL,),
        in_specs=[
            pl.BlockSpec((D, B), lambda j: (0, 0)),
            pl.BlockSpec((D, D), lambda j: (0, 0)),
            pl.BlockSpec((D, 1), lambda j: (0, 0)),
            pl.BlockSpec((D, S_BLK), lambda j: (0, j)),
            pl.BlockSpec((D, S_BLK), lambda j: (0, j)),
            pl.BlockSpec((D, S_BLK), lambda j: (0, NFULL)),
            pl.BlockSpec((D, S_BLK), lambda j: (0, NFULL)),
        ],
        out_specs=[
            pl.BlockSpec((D, B), lambda j: (0, 0)),
            pl.BlockSpec((1, B), lambda j: (0, 0)),
            pl.BlockSpec((1, B), lambda j: (0, 0)),
            pl.BlockSpec((1, B), lambda j: (0, 0)),
            pl.BlockSpec((D, B), lambda j: (0, 0)),
        ],
        out_shape=[
            jax.ShapeDtypeStruct((D, B), jnp.float32),
            jax.ShapeDtypeStruct((1, B), jnp.float32),
            jax.ShapeDtypeStruct((1, B), jnp.int32),
            jax.ShapeDtypeStruct((1, B), jnp.float32),
            jax.ShapeDtypeStruct((D, B), jnp.float32),
        ],
        scratch_shapes=[
            pltpu.VMEM((D + 1, B), jnp.float32),
            pltpu.VMEM((1, B), jnp.float32),
            pltpu.VMEM((1, B), jnp.int32),
        ],
        compiler_params=pltpu.CompilerParams(
            dimension_semantics=("arbitrary",),
        ),
    )(qt, wt, bt, kt, vt, kt, vt)

    attn_t, nv_t = pl.pallas_call(
        _write_kernel,
        grid=(NS,),
        in_specs=[
            pl.BlockSpec((D, B), lambda j: (0, 0)),
            pl.BlockSpec((1, B), lambda j: (0, 0)),
            pl.BlockSpec((B, 1), lambda j: (0, 0)),
            pl.BlockSpec((1, B), lambda j: (0, 0)),
            pl.BlockSpec((D, B), lambda j: (0, 0)),
            pl.BlockSpec((1, 1), lambda j: (0, 0)),
            pl.BlockSpec((D, S_BLK), lambda j: (0, j)),
            pl.BlockSpec((D, S_BLK), lambda j: (0, j)),
        ],
        out_specs=[
            pl.BlockSpec((S_BLK, B), lambda j: (j, 0)),
            pl.BlockSpec((D, S_BLK), lambda j: (0, j)),
        ],
        out_shape=[
            jax.ShapeDtypeStruct((S, B), jnp.float32),
            jax.ShapeDtypeStruct((D, S), jnp.float32),
        ],
        scratch_shapes=[
            pltpu.VMEM((1, B), jnp.float32),
            pltpu.VMEM((D + 1, B), jnp.float32),
        ],
        compiler_params=pltpu.CompilerParams(
            dimension_semantics=("arbitrary",),
        ),
    )(qp_t, invl_t, bi_t.reshape(B, 1), bi_t, wvt, wr, kt, vt)

    return retr_t.T, attn_t.T, energy_t.reshape(B), nv_t.T
